# baseline (speedup 1.0000x reference)
import functools

import jax
import jax.numpy as jnp
from jax import lax
from jax.experimental import pallas as pl
from jax.experimental.pallas import tpu as pltpu
from jax.experimental.pallas import tpu_sc as plsc

N_BINS = 256
OUT_DIM = 256
L = 16
NW = 32
B = 16 * 4096
B_W = B // NW
CHUNK = 64
NCH = B_W // CHUNK
PAD_LO = 8
PBINS = 288
NBUF = 2


def _sc_kernel():
    mesh = plsc.VectorSubcoreMesh(core_axis_name="c", subcore_axis_name="s")

    @functools.partial(
        pl.kernel,
        mesh=mesh,
        out_type=jax.ShapeDtypeStruct((B, OUT_DIM), jnp.float32),
        compiler_params=pltpu.CompilerParams(needs_layout_passes=False),
        scratch_types=[
            pltpu.VMEM((B_W // L, L), jnp.float32),
            pltpu.VMEM((PBINS,), jnp.float32),
            pltpu.VMEM((2, L), jnp.float32),
            pltpu.VMEM((N_BINS, OUT_DIM), jnp.float32),
            pltpu.VMEM((NBUF * CHUNK, OUT_DIM), jnp.float32),
            pltpu.SemaphoreType.DMA,
        ],
    )
    def k(x_hbm, pbins_hbm, consts_hbm, embt_hbm, out_hbm,
          xv, pbinsv, constsv, embv, bufv, wsem):
        wid = lax.axis_index("s") * 2 + lax.axis_index("c")
        base = wid * B_W

        pltpu.sync_copy(x_hbm.at[wid], xv)
        pltpu.sync_copy(pbins_hbm, pbinsv)
        pltpu.sync_copy(consts_hbm, constsv)
        pltpu.sync_copy(embt_hbm.at[pl.ds(wid * N_BINS, N_BINS)], embv)

        phi0 = constsv[0]
        inv_s = constsv[1]

        def chunk_body(c, _):
            bslot = lax.rem(c, NBUF)

            def group_body(g, _):
                xvec = xv[c * (CHUNK // L) + g]
                xi = lax.bitcast_convert_type(xvec, jnp.int32)
                gf = (xi.astype(jnp.float32) - phi0) * inv_s + 0.5
                gs = gf.astype(jnp.int32)
                gs = jnp.minimum(jnp.maximum(gs, 0), N_BINS - 1)
                cnt = gs - 3
                for kk in range(6):
                    bv = plsc.load_gather(pbinsv, [gs + (PAD_LO - 3 + kk)])
                    cnt = cnt + jnp.where(bv < xvec, 1, 0)
                dbase = bslot * CHUNK + g * L
                for i in range(L):
                    s = cnt[i]
                    for t in range(OUT_DIM // L):
                        bufv[dbase + i, pl.ds(t * L, L)] = \
                            embv[s, pl.ds(t * L, L)]
                return _

            lax.fori_loop(0, CHUNK // L, group_body, None)

            @pl.when(c >= NBUF)
            def _wait():
                pltpu.make_async_copy(
                    bufv.at[pl.ds(bslot * CHUNK, CHUNK)],
                    out_hbm.at[pl.ds(base + c * CHUNK, CHUNK)],
                    wsem).wait()

            pltpu.async_copy(
                bufv.at[pl.ds(bslot * CHUNK, CHUNK)],
                out_hbm.at[pl.ds(base + c * CHUNK, CHUNK)],
                wsem)
            return _

        lax.fori_loop(0, NCH, chunk_body, None)

        for b in range(min(NBUF, NCH)):
            pltpu.make_async_copy(
                bufv.at[pl.ds(b * CHUNK, CHUNK)],
                out_hbm.at[pl.ds(base, CHUNK)],
                wsem).wait()

    return k


def kernel(x, energy_bins, emb):
    x3 = x.reshape(NW, B_W // L, L)
    pbins = jnp.concatenate([
        jnp.full((PAD_LO,), -1e38, jnp.float32),
        energy_bins,
        jnp.full((PBINS - PAD_LO - (N_BINS - 1),), 1e38, jnp.float32),
    ])
    bi = lax.bitcast_convert_type(energy_bins, jnp.int32)
    phi0 = bi[0].astype(jnp.float32)
    inv_s = jnp.float32(N_BINS - 2) / (bi[N_BINS - 2].astype(jnp.float32) - phi0)
    consts = jnp.stack([jnp.full((L,), phi0), jnp.full((L,), inv_s)])
    embt = jnp.tile(emb, (NW, 1))
    out = _sc_kernel()(x3, pbins, consts, embt)
    return out.reshape(x.shape[0], x.shape[1], OUT_DIM)


# speedup vs baseline: 1.9663x; 1.9663x over previous
import functools

import jax
import jax.numpy as jnp
from jax import lax
from jax.experimental import pallas as pl
from jax.experimental.pallas import tpu as pltpu
from jax.experimental.pallas import tpu_sc as plsc

N_BINS = 256
OUT_DIM = 256
L = 16
NW = 32
B = 16 * 4096
B_W = B // NW
CHUNK = 64
NCH = B_W // CHUNK
PAD_LO = 8
PBINS = 288
NBUF = 2


def _sc_kernel():
    mesh = plsc.VectorSubcoreMesh(core_axis_name="c", subcore_axis_name="s")

    @functools.partial(
        pl.kernel,
        mesh=mesh,
        out_type=jax.ShapeDtypeStruct((B, OUT_DIM), jnp.float32),
        compiler_params=pltpu.CompilerParams(needs_layout_passes=False),
        scratch_types=[
            pltpu.VMEM((B_W // L, L), jnp.float32),
            pltpu.VMEM((PBINS,), jnp.float32),
            pltpu.VMEM((2, L), jnp.float32),
            pltpu.VMEM((N_BINS, OUT_DIM), jnp.float32),
            pltpu.VMEM((NBUF * CHUNK, OUT_DIM), jnp.float32),
            pltpu.SemaphoreType.DMA,
        ],
    )
    def k(x_hbm, pbins_hbm, consts_hbm, embt_hbm, out_hbm,
          xv, pbinsv, constsv, embv, bufv, wsem):
        wid = lax.axis_index("s") * 2 + lax.axis_index("c")
        base = wid * B_W

        pltpu.sync_copy(x_hbm.at[wid], xv)
        pltpu.sync_copy(pbins_hbm, pbinsv)
        pltpu.sync_copy(consts_hbm, constsv)
        pltpu.sync_copy(embt_hbm.at[pl.ds(wid * N_BINS, N_BINS)], embv)

        phi0 = constsv[0]
        inv_s = constsv[1]

        def chunk_body(c, _):
            bslot = lax.rem(c, NBUF)

            def group_body(g, _):
                xvec = xv[c * (CHUNK // L) + g]
                xi = lax.bitcast_convert_type(xvec, jnp.int32)
                gf = (xi.astype(jnp.float32) - phi0) * inv_s + 0.5
                gs = gf.astype(jnp.int32)
                gs = jnp.minimum(jnp.maximum(gs, 0), N_BINS - 1)
                cnt = gs - 3
                for kk in range(6):
                    bv = plsc.load_gather(pbinsv, [gs + (PAD_LO - 3 + kk)])
                    cnt = cnt + jnp.where(bv < xvec, 1, 0)
                dbase = bslot * CHUNK + g * L
                for i in range(L):
                    s = cnt[i]
                    vals = [embv[s, pl.ds(t * L, L)]
                            for t in range(OUT_DIM // L)]
                    for t in range(OUT_DIM // L):
                        bufv[dbase + i, pl.ds(t * L, L)] = vals[t]
                return _

            lax.fori_loop(0, CHUNK // L, group_body, None)

            @pl.when(c >= NBUF)
            def _wait():
                pltpu.make_async_copy(
                    bufv.at[pl.ds(bslot * CHUNK, CHUNK)],
                    out_hbm.at[pl.ds(base + c * CHUNK, CHUNK)],
                    wsem).wait()

            pltpu.async_copy(
                bufv.at[pl.ds(bslot * CHUNK, CHUNK)],
                out_hbm.at[pl.ds(base + c * CHUNK, CHUNK)],
                wsem)
            return _

        lax.fori_loop(0, NCH, chunk_body, None)

        for b in range(min(NBUF, NCH)):
            pltpu.make_async_copy(
                bufv.at[pl.ds(b * CHUNK, CHUNK)],
                out_hbm.at[pl.ds(base, CHUNK)],
                wsem).wait()

    return k


def kernel(x, energy_bins, emb):
    x3 = x.reshape(NW, B_W // L, L)
    pbins = jnp.concatenate([
        jnp.full((PAD_LO,), -1e38, jnp.float32),
        energy_bins,
        jnp.full((PBINS - PAD_LO - (N_BINS - 1),), 1e38, jnp.float32),
    ])
    bi = lax.bitcast_convert_type(energy_bins, jnp.int32)
    phi0 = bi[0].astype(jnp.float32)
    inv_s = jnp.float32(N_BINS - 2) / (bi[N_BINS - 2].astype(jnp.float32) - phi0)
    consts = jnp.stack([jnp.full((L,), phi0), jnp.full((L,), inv_s)])
    embt = jnp.tile(emb, (NW, 1))
    out = _sc_kernel()(x3, pbins, consts, embt)
    return out.reshape(x.shape[0], x.shape[1], OUT_DIM)


# speedup vs baseline: 2.0360x; 1.0355x over previous
import functools

import jax
import jax.numpy as jnp
from jax import lax
from jax.experimental import pallas as pl
from jax.experimental.pallas import tpu as pltpu
from jax.experimental.pallas import tpu_sc as plsc

N_BINS = 256
OUT_DIM = 256
L = 16
NW = 32
B = 16 * 4096
B_W = B // NW
CHUNK = 64
NCH = B_W // CHUNK
PAD_LO = 8
PBINS = 288
NBUF = 2


def _sc_kernel():
    mesh = plsc.VectorSubcoreMesh(core_axis_name="c", subcore_axis_name="s")

    @functools.partial(
        pl.kernel,
        mesh=mesh,
        out_type=jax.ShapeDtypeStruct((B, OUT_DIM), jnp.float32),
        compiler_params=pltpu.CompilerParams(needs_layout_passes=False),
        scratch_types=[
            pltpu.VMEM((B_W // L, L), jnp.float32),
            pltpu.VMEM((PBINS,), jnp.float32),
            pltpu.VMEM((2, L), jnp.float32),
            pltpu.VMEM((N_BINS, OUT_DIM), jnp.float32),
            pltpu.VMEM((NBUF * CHUNK, OUT_DIM), jnp.float32),
            pltpu.SemaphoreType.DMA,
        ],
    )
    def k(x_hbm, pbins_hbm, consts_hbm, emb_hbm, out_hbm,
          xv, pbinsv, constsv, embv, bufv, wsem):
        wid = lax.axis_index("s") * 2 + lax.axis_index("c")
        base = wid * B_W

        pltpu.sync_copy(x_hbm.at[wid], xv)
        pltpu.sync_copy(pbins_hbm, pbinsv)
        pltpu.sync_copy(consts_hbm, constsv)
        pltpu.sync_copy(emb_hbm, embv)

        phi0 = constsv[0]
        inv_s = constsv[1]

        def chunk_body(c, _):
            bslot = lax.rem(c, NBUF)

            def group_body(g, _):
                xvec = xv[c * (CHUNK // L) + g]
                xi = lax.bitcast_convert_type(xvec, jnp.int32)
                gf = (xi.astype(jnp.float32) - phi0) * inv_s + 0.5
                gs = gf.astype(jnp.int32)
                gs = jnp.minimum(jnp.maximum(gs, 0), N_BINS - 1)
                cnt = gs - 3
                for kk in range(6):
                    bv = plsc.load_gather(pbinsv, [gs + (PAD_LO - 3 + kk)])
                    cnt = cnt + jnp.where(bv < xvec, 1, 0)
                dbase = bslot * CHUNK + g * L
                for i in range(L):
                    s = cnt[i]
                    vals = [embv[s, pl.ds(t * L, L)]
                            for t in range(OUT_DIM // L)]
                    for t in range(OUT_DIM // L):
                        bufv[dbase + i, pl.ds(t * L, L)] = vals[t]
                return _

            lax.fori_loop(0, CHUNK // L, group_body, None)

            @pl.when(c >= NBUF)
            def _wait():
                pltpu.make_async_copy(
                    bufv.at[pl.ds(bslot * CHUNK, CHUNK)],
                    out_hbm.at[pl.ds(base + c * CHUNK, CHUNK)],
                    wsem).wait()

            pltpu.async_copy(
                bufv.at[pl.ds(bslot * CHUNK, CHUNK)],
                out_hbm.at[pl.ds(base + c * CHUNK, CHUNK)],
                wsem)
            return _

        lax.fori_loop(0, NCH, chunk_body, None)

        for b in range(min(NBUF, NCH)):
            pltpu.make_async_copy(
                bufv.at[pl.ds(b * CHUNK, CHUNK)],
                out_hbm.at[pl.ds(base, CHUNK)],
                wsem).wait()

    return k


def kernel(x, energy_bins, emb):
    x3 = x.reshape(NW, B_W // L, L)
    pbins = jnp.concatenate([
        jnp.full((PAD_LO,), -1e38, jnp.float32),
        energy_bins,
        jnp.full((PBINS - PAD_LO - (N_BINS - 1),), 1e38, jnp.float32),
    ])
    bi = lax.bitcast_convert_type(energy_bins, jnp.int32)
    phi0 = bi[0].astype(jnp.float32)
    inv_s = jnp.float32(N_BINS - 2) / (bi[N_BINS - 2].astype(jnp.float32) - phi0)
    consts = jnp.stack([jnp.full((L,), phi0), jnp.full((L,), inv_s)])
    out = _sc_kernel()(x3, pbins, consts, emb)
    return out.reshape(x.shape[0], x.shape[1], OUT_DIM)


# speedup vs baseline: 2.2214x; 1.0910x over previous
import functools

import jax
import jax.numpy as jnp
from jax import lax
from jax.experimental import pallas as pl
from jax.experimental.pallas import tpu as pltpu
from jax.experimental.pallas import tpu_sc as plsc

N_BINS = 256
OUT_DIM = 256
L = 16
NW = 32
B = 16 * 4096
B_W = B // NW
CHUNK = 64
NCH = B_W // CHUNK
PAD_LO = 8
PBINS = 288
NBUF = 2


def _sc_kernel():
    mesh = plsc.VectorSubcoreMesh(core_axis_name="c", subcore_axis_name="s")

    @functools.partial(
        pl.kernel,
        mesh=mesh,
        out_type=jax.ShapeDtypeStruct((B, OUT_DIM), jnp.float32),
        compiler_params=pltpu.CompilerParams(needs_layout_passes=False),
        scratch_types=[
            pltpu.VMEM((B_W // L, L), jnp.float32),
            pltpu.VMEM((PBINS,), jnp.float32),
            pltpu.VMEM((N_BINS, OUT_DIM), jnp.float32),
            pltpu.VMEM((NBUF * CHUNK, OUT_DIM), jnp.float32),
            pltpu.SemaphoreType.DMA,
            pltpu.SemaphoreType.DMA,
        ],
    )
    def k(x_hbm, bins_hbm, emb_hbm, out_hbm,
          xv, pbinsv, embv, bufv, ssem, wsem):
        wid = lax.axis_index("s") * 2 + lax.axis_index("c")
        base = wid * B_W

        stage = pltpu.async_copy(emb_hbm, embv, ssem)
        pltpu.sync_copy(x_hbm.at[wid], xv)

        pbinsv[pl.ds(0, L)] = jnp.full((L,), -1e38, jnp.float32)
        pbinsv[pl.ds(N_BINS, L)] = jnp.full((L,), 1e38, jnp.float32)
        pbinsv[pl.ds(N_BINS + L, L)] = jnp.full((L,), 1e38, jnp.float32)
        pltpu.sync_copy(bins_hbm, pbinsv.at[pl.ds(PAD_LO, N_BINS - 1)])

        f_lo = lax.bitcast_convert_type(
            pbinsv[pl.ds(PAD_LO, L)], jnp.int32).astype(jnp.float32)
        f_hi = lax.bitcast_convert_type(
            pbinsv[pl.ds(PAD_LO + 240, L)], jnp.int32).astype(jnp.float32)
        phi0 = jnp.full((L,), f_lo[0], jnp.float32)
        inv_s = jnp.full((L,), jnp.float32(N_BINS - 2), jnp.float32) / (
            jnp.full((L,), f_hi[14], jnp.float32) - phi0)

        def bucketize_row(r, _):
            xvec = xv[r]
            xi = lax.bitcast_convert_type(xvec, jnp.int32)
            gf = (xi.astype(jnp.float32) - phi0) * inv_s + 0.5
            gs = gf.astype(jnp.int32)
            gs = jnp.minimum(jnp.maximum(gs, 0), N_BINS - 1)
            cnt = gs - 3
            for kk in range(6):
                bv = plsc.load_gather(pbinsv, [gs + (PAD_LO - 3 + kk)])
                cnt = cnt + jnp.where(bv < xvec, 1, 0)
            xv[r] = lax.bitcast_convert_type(cnt, jnp.float32)
            return _

        lax.fori_loop(0, B_W // L, bucketize_row, None)
        stage.wait()

        def chunk_body(c, _):
            bslot = lax.rem(c, NBUF)

            def group_body(g, _):
                vidx = lax.bitcast_convert_type(
                    xv[c * (CHUNK // L) + g], jnp.int32)
                dbase = bslot * CHUNK + g * L
                for i in range(L):
                    s = vidx[i]
                    vals = [embv[s, pl.ds(t * L, L)]
                            for t in range(OUT_DIM // L)]
                    for t in range(OUT_DIM // L):
                        bufv[dbase + i, pl.ds(t * L, L)] = vals[t]
                return _

            lax.fori_loop(0, CHUNK // L, group_body, None)

            @pl.when(c >= NBUF)
            def _wait():
                pltpu.make_async_copy(
                    bufv.at[pl.ds(bslot * CHUNK, CHUNK)],
                    out_hbm.at[pl.ds(base + c * CHUNK, CHUNK)],
                    wsem).wait()

            pltpu.async_copy(
                bufv.at[pl.ds(bslot * CHUNK, CHUNK)],
                out_hbm.at[pl.ds(base + c * CHUNK, CHUNK)],
                wsem)
            return _

        lax.fori_loop(0, NCH, chunk_body, None)

        for b in range(min(NBUF, NCH)):
            pltpu.make_async_copy(
                bufv.at[pl.ds(b * CHUNK, CHUNK)],
                out_hbm.at[pl.ds(base, CHUNK)],
                wsem).wait()

    return k


def kernel(x, energy_bins, emb):
    x3 = x.reshape(NW, B_W // L, L)
    out = _sc_kernel()(x3, energy_bins, emb)
    return out.reshape(x.shape[0], x.shape[1], OUT_DIM)


# speedup vs baseline: 2.2254x; 1.0018x over previous
import functools

import jax
import jax.numpy as jnp
from jax import lax
from jax.experimental import pallas as pl
from jax.experimental.pallas import tpu as pltpu
from jax.experimental.pallas import tpu_sc as plsc

N_BINS = 256
OUT_DIM = 256
L = 16
NW = 32
B = 16 * 4096
B_W = B // NW
CHUNK = 64
NCH = B_W // CHUNK
PAD_LO = 8
PBINS = 288
NBUF = 2


def _sc_kernel():
    mesh = plsc.VectorSubcoreMesh(core_axis_name="c", subcore_axis_name="s")

    @functools.partial(
        pl.kernel,
        mesh=mesh,
        out_type=jax.ShapeDtypeStruct((B, OUT_DIM), jnp.float32),
        compiler_params=pltpu.CompilerParams(needs_layout_passes=False),
        scratch_types=[
            pltpu.VMEM((B_W // L, L), jnp.float32),
            pltpu.VMEM((PBINS,), jnp.float32),
            pltpu.VMEM((N_BINS, OUT_DIM), jnp.float32),
            pltpu.VMEM((NBUF * CHUNK, OUT_DIM), jnp.float32),
            pltpu.SemaphoreType.DMA,
            pltpu.SemaphoreType.DMA,
        ],
    )
    def k(x_hbm, bins_hbm, emb_hbm, out_hbm,
          xv, pbinsv, embv, bufv, ssem, wsem):
        wid = lax.axis_index("s") * 2 + lax.axis_index("c")
        base = wid * B_W

        stage = pltpu.async_copy(emb_hbm, embv, ssem)
        pltpu.sync_copy(x_hbm.at[wid], xv)

        pbinsv[pl.ds(0, L)] = jnp.full((L,), -1e38, jnp.float32)
        pbinsv[pl.ds(N_BINS, L)] = jnp.full((L,), 1e38, jnp.float32)
        pbinsv[pl.ds(N_BINS + L, L)] = jnp.full((L,), 1e38, jnp.float32)
        pltpu.sync_copy(bins_hbm, pbinsv.at[pl.ds(PAD_LO, N_BINS - 1)])

        f_lo = lax.bitcast_convert_type(
            pbinsv[pl.ds(PAD_LO, L)], jnp.int32).astype(jnp.float32)
        f_hi = lax.bitcast_convert_type(
            pbinsv[pl.ds(PAD_LO + 240, L)], jnp.int32).astype(jnp.float32)
        phi0 = jnp.full((L,), f_lo[0], jnp.float32)
        inv_s = jnp.full((L,), jnp.float32(N_BINS - 2), jnp.float32) / (
            jnp.full((L,), f_hi[14], jnp.float32) - phi0)

        def bucketize_row(r, _):
            xvec = xv[r]
            xi = lax.bitcast_convert_type(xvec, jnp.int32)
            gf = (xi.astype(jnp.float32) - phi0) * inv_s + 0.5
            gs = gf.astype(jnp.int32)
            gs = jnp.minimum(jnp.maximum(gs, 0), N_BINS - 1)
            cnt = gs - 3
            for kk in range(6):
                bv = plsc.load_gather(pbinsv, [gs + (PAD_LO - 3 + kk)])
                cnt = cnt + jnp.where(bv < xvec, 1, 0)
            xv[r] = lax.bitcast_convert_type(cnt, jnp.float32)
            return _

        lax.fori_loop(0, B_W // L, bucketize_row, None)
        stage.wait()

        def chunk_body(c, _):
            bslot = lax.rem(c, NBUF)

            def group_body(g, _):
                vidx = lax.bitcast_convert_type(
                    xv[c * (CHUNK // L) + g], jnp.int32)
                dbase = bslot * CHUNK + g * L
                rows = [vidx[i] for i in range(L)]
                for i in range(L):
                    s = rows[i]
                    vals = [embv[s, pl.ds(t * L, L)]
                            for t in range(OUT_DIM // L)]
                    for t in range(OUT_DIM // L):
                        bufv[dbase + i, pl.ds(t * L, L)] = vals[t]
                return _

            lax.fori_loop(0, CHUNK // L, group_body, None)

            @pl.when(c >= NBUF)
            def _wait():
                pltpu.make_async_copy(
                    bufv.at[pl.ds(bslot * CHUNK, CHUNK)],
                    out_hbm.at[pl.ds(base + c * CHUNK, CHUNK)],
                    wsem).wait()

            pltpu.async_copy(
                bufv.at[pl.ds(bslot * CHUNK, CHUNK)],
                out_hbm.at[pl.ds(base + c * CHUNK, CHUNK)],
                wsem)
            return _

        lax.fori_loop(0, NCH, chunk_body, None)

        for b in range(min(NBUF, NCH)):
            pltpu.make_async_copy(
                bufv.at[pl.ds(b * CHUNK, CHUNK)],
                out_hbm.at[pl.ds(base, CHUNK)],
                wsem).wait()

    return k


def kernel(x, energy_bins, emb):
    x3 = x.reshape(NW, B_W // L, L)
    out = _sc_kernel()(x3, energy_bins, emb)
    return out.reshape(x.shape[0], x.shape[1], OUT_DIM)
